# baseline (speedup 1.0000x reference)
import functools

import jax
import jax.numpy as jnp
from jax import lax
from jax.experimental import pallas as pl
from jax.experimental.pallas import tpu as pltpu
from jax.experimental.pallas import tpu_sc as plsc

NUM_EMB = 100000
DIM = 64
PDIM = 128
L = 50
B = 4096
NC = 2
NS = 16
NW = NC * NS
CHUNK = 128
NBUF = 4
NTASK_PAD = 52
E_BLK = 2048

_mesh = plsc.VectorSubcoreMesh(core_axis_name="c", subcore_axis_name="s")


def _tp_body(tt_ref, o_ref):
    o_ref[:, 0:DIM] = tt_ref[...].T


_transpose_table = pl.pallas_call(
    _tp_body,
    grid=(pl.cdiv(NUM_EMB, E_BLK),),
    in_specs=[pl.BlockSpec((DIM, E_BLK), lambda i: (0, i))],
    out_specs=pl.BlockSpec((E_BLK, PDIM), lambda i: (i, 0)),
    out_shape=jax.ShapeDtypeStruct((NUM_EMB, PDIM), jnp.float32),
)


@functools.partial(
    pl.kernel,
    mesh=_mesh,
    out_type=jax.ShapeDtypeStruct((L, DIM // 8, B // CHUNK, 8, CHUNK), jnp.float32),
    scratch_types=[
        pltpu.VMEM((L, CHUNK), jnp.int32),
        pltpu.VMEM((NBUF, CHUNK, PDIM), jnp.float32),
        pltpu.VMEM((NBUF, DIM, CHUNK), jnp.float32),
        pltpu.VMEM((8, CHUNK), jnp.float32),
    ]
    + [pltpu.SemaphoreType.DMA] * (1 + 2 * NBUF),
    compiler_params=pltpu.CompilerParams(
        use_tc_tiling_on_sc=False, needs_layout_passes=False
    ),
)
def _emb_fused(xt_hbm, table_hbm, out_hbm, idx_v, gbuf, tbuf, dump, *sems):
    idx_sem = sems[0]
    gsems = sems[1 : 1 + NBUF]
    osems = sems[1 + NBUF :]
    wid = lax.axis_index("s") * NC + lax.axis_index("c")

    pltpu.async_copy(
        xt_hbm.at[:, pl.ds(wid * CHUNK, CHUNK)], idx_v, idx_sem
    ).wait()

    lanes16 = lax.iota(jnp.int32, 16)
    rows_g = [g * 16 + lanes16 for g in range(CHUNK // 16)]
    last = jnp.int32(L - 1)

    def _start_gather(t, b):
        return pltpu.async_copy(
            table_hbm.at[idx_v.at[t]], gbuf.at[b], gsems[b]
        )

    def _wait_gather(b):
        pltpu.make_async_copy(
            table_hbm.at[pl.ds(0, CHUNK)], gbuf.at[b], gsems[b]
        ).wait()

    def _put(l, b):
        for fg in range(DIM // 8):
            pltpu.async_copy(
                tbuf.at[b, pl.ds(8 * fg, 8)],
                out_hbm.at[l, fg, wid],
                osems[b],
            )

    def _wait_puts(b):
        for fg in range(DIM // 8):
            pltpu.make_async_copy(
                tbuf.at[b, pl.ds(8 * fg, 8)],
                out_hbm.at[0, fg, wid],
                osems[b],
            ).wait()

    for b in range(NBUF):
        for fg in range(DIM // 8):
            pltpu.async_copy(out_hbm.at[0, fg, wid], dump, osems[b])

    for t in range(NBUF):
        _start_gather(jnp.int32(t), t)

    def trip(i, carry):
        for b in range(NBUF):
            l_raw = i * NBUF + b
            l = jnp.minimum(l_raw, last)
            _wait_gather(b)
            _wait_puts(b)

            mfs = []
            for g in range(CHUNK // 16):
                v = idx_v[l, pl.ds(g * 16, 16)]
                mfs.append(jnp.where(v == 0, 0.0, 1.0).astype(jnp.float32))

            def fbody(fq, c2, b=b, mfs=mfs):
                for df in range(8):
                    f = fq * 8 + df
                    col = jnp.full((16,), f, jnp.int32)
                    for g in range(CHUNK // 16):
                        vals = plsc.load_gather(gbuf.at[b], [rows_g[g], col])
                        tbuf[b, f, pl.ds(g * 16, 16)] = vals * mfs[g]
                return c2

            lax.fori_loop(0, DIM // 8, fbody, 0, unroll=False)

            _start_gather(jnp.minimum(l + NBUF, last), b)
            _put(l, b)
        return carry

    lax.fori_loop(0, NTASK_PAD // NBUF, trip, 0, unroll=False)

    for b in range(NBUF):
        _wait_gather(b)
        _wait_puts(b)


def kernel(x, table):
    xt = x.T.astype(jnp.int32)
    tablep = _transpose_table(table.T)
    out5 = _emb_fused(xt, tablep)
    return out5.transpose(2, 4, 0, 1, 3).reshape(B, L, DIM)


# speedup vs baseline: 1.4095x; 1.4095x over previous
import functools

import jax
import jax.numpy as jnp
from jax import lax
from jax.experimental import pallas as pl
from jax.experimental.pallas import tpu as pltpu
from jax.experimental.pallas import tpu_sc as plsc

NUM_EMB = 100000
DIM = 64
PDIM = 128
L = 50
B = 4096
NC = 2
NS = 16
NW = NC * NS
CHUNK = 128
NBUF = 4
NTASK_PAD = 52
E_BLK = 2048

_mesh = plsc.VectorSubcoreMesh(core_axis_name="c", subcore_axis_name="s")


def _tp_body(tt_ref, o_ref):
    o_ref[:, 0:DIM] = tt_ref[...].T


_transpose_table = pl.pallas_call(
    _tp_body,
    grid=(pl.cdiv(NUM_EMB, E_BLK),),
    in_specs=[pl.BlockSpec((DIM, E_BLK), lambda i: (0, i))],
    out_specs=pl.BlockSpec((E_BLK, PDIM), lambda i: (i, 0)),
    out_shape=jax.ShapeDtypeStruct((NUM_EMB, PDIM), jnp.float32),
)


@functools.partial(
    pl.kernel,
    mesh=_mesh,
    out_type=jax.ShapeDtypeStruct((L, DIM // 8, B // CHUNK, 8, CHUNK), jnp.float32),
    scratch_types=[
        pltpu.VMEM((L, CHUNK), jnp.int32),
        pltpu.VMEM((NBUF, CHUNK, PDIM), jnp.float32),
        pltpu.VMEM((NBUF, DIM, CHUNK), jnp.float32),
        pltpu.VMEM((8, CHUNK), jnp.float32),
    ]
    + [pltpu.SemaphoreType.DMA] * (1 + 2 * NBUF),
    compiler_params=pltpu.CompilerParams(
        use_tc_tiling_on_sc=False, needs_layout_passes=False
    ),
)
def _emb_fused(xt_hbm, table_hbm, out_hbm, idx_v, gbuf, tbuf, dump, *sems):
    idx_sem = sems[0]
    gsems = sems[1 : 1 + NBUF]
    osems = sems[1 + NBUF :]
    wid = lax.axis_index("s") * NC + lax.axis_index("c")

    pltpu.async_copy(
        xt_hbm.at[:, pl.ds(wid * CHUNK, CHUNK)], idx_v, idx_sem
    ).wait()

    lanes16 = lax.iota(jnp.int32, 16)
    rows_g = [g * 16 + lanes16 for g in range(CHUNK // 16)]
    last = jnp.int32(L - 1)

    def _start_gather(t, b):
        return pltpu.async_copy(
            table_hbm.at[idx_v.at[t]], gbuf.at[b], gsems[b]
        )

    def _wait_gather(b):
        pltpu.make_async_copy(
            table_hbm.at[pl.ds(0, CHUNK)], gbuf.at[b], gsems[b]
        ).wait()

    def _put(l, b):
        for fg in range(DIM // 8):
            pltpu.async_copy(
                tbuf.at[b, pl.ds(8 * fg, 8)],
                out_hbm.at[l, fg, wid],
                osems[b],
            )

    def _wait_puts(b):
        for fg in range(DIM // 8):
            pltpu.make_async_copy(
                tbuf.at[b, pl.ds(8 * fg, 8)],
                out_hbm.at[0, fg, wid],
                osems[b],
            ).wait()

    for b in range(NBUF):
        for fg in range(DIM // 8):
            pltpu.async_copy(out_hbm.at[0, fg, wid], dump, osems[b])

    for t in range(NBUF):
        _start_gather(jnp.int32(t), t)

    def trip(i, carry):
        for b in range(NBUF):
            l_raw = i * NBUF + b
            l = jnp.minimum(l_raw, last)
            _wait_gather(b)
            _wait_puts(b)

            mfs = []
            for g in range(CHUNK // 16):
                v = idx_v[l, pl.ds(g * 16, 16)]
                mfs.append(jnp.where(v == 0, 0.0, 1.0).astype(jnp.float32))

            @plsc.parallel_loop(0, DIM // 8, unroll=2)
            def fbody(fq, b=b, mfs=mfs):
                for df in range(8):
                    f = fq * 8 + df
                    col = jnp.full((16,), f, jnp.int32)
                    vals = [
                        plsc.load_gather(gbuf.at[b], [rows_g[g], col])
                        for g in range(CHUNK // 16)
                    ]
                    for g in range(CHUNK // 16):
                        tbuf[b, f, pl.ds(g * 16, 16)] = vals[g] * mfs[g]

            _start_gather(jnp.minimum(l + NBUF, last), b)
            _put(l, b)
        return carry

    lax.fori_loop(0, NTASK_PAD // NBUF, trip, 0, unroll=False)

    for b in range(NBUF):
        _wait_gather(b)
        _wait_puts(b)


def kernel(x, table):
    xt = x.T.astype(jnp.int32)
    tablep = _transpose_table(table.T)
    out5 = _emb_fused(xt, tablep)
    return out5.transpose(2, 4, 0, 1, 3).reshape(B, L, DIM)


# speedup vs baseline: 1.4241x; 1.0104x over previous
import functools

import jax
import jax.numpy as jnp
from jax import lax
from jax.experimental import pallas as pl
from jax.experimental.pallas import tpu as pltpu
from jax.experimental.pallas import tpu_sc as plsc

NUM_EMB = 100000
DIM = 64
PDIM = 128
L = 50
B = 4096
NC = 2
NS = 16
NW = NC * NS
CHUNK = 128
NBUF = 4
NTASK_PAD = 52
E_BLK = 2048

_mesh = plsc.VectorSubcoreMesh(core_axis_name="c", subcore_axis_name="s")


def _tp_body(tt_ref, o_ref):
    o_ref[:, 0:DIM] = tt_ref[...].T


_transpose_table = pl.pallas_call(
    _tp_body,
    grid=(pl.cdiv(NUM_EMB, E_BLK),),
    in_specs=[pl.BlockSpec((DIM, E_BLK), lambda i: (0, i))],
    out_specs=pl.BlockSpec((E_BLK, PDIM), lambda i: (i, 0)),
    out_shape=jax.ShapeDtypeStruct((NUM_EMB, PDIM), jnp.float32),
)


@functools.partial(
    pl.kernel,
    mesh=_mesh,
    out_type=jax.ShapeDtypeStruct((L, DIM // 8, B // CHUNK, 8, CHUNK), jnp.float32),
    scratch_types=[
        pltpu.VMEM((L, CHUNK), jnp.int32),
        pltpu.VMEM((NBUF, CHUNK, PDIM), jnp.float32),
        pltpu.VMEM((NBUF, DIM // 8, 8, CHUNK), jnp.float32),
        pltpu.VMEM((DIM // 8, 8, CHUNK), jnp.float32),
    ]
    + [pltpu.SemaphoreType.DMA] * (1 + 2 * NBUF),
    compiler_params=pltpu.CompilerParams(
        use_tc_tiling_on_sc=False, needs_layout_passes=False
    ),
)
def _emb_fused(xt_hbm, table_hbm, out_hbm, idx_v, gbuf, tbuf, dump, *sems):
    idx_sem = sems[0]
    gsems = sems[1 : 1 + NBUF]
    osems = sems[1 + NBUF :]
    wid = lax.axis_index("s") * NC + lax.axis_index("c")

    pltpu.async_copy(
        xt_hbm.at[:, pl.ds(wid * CHUNK, CHUNK)], idx_v, idx_sem
    ).wait()

    lanes16 = lax.iota(jnp.int32, 16)
    rows_g = [g * 16 + lanes16 for g in range(CHUNK // 16)]
    last = jnp.int32(L - 1)

    def _start_gather(t, b):
        return pltpu.async_copy(
            table_hbm.at[idx_v.at[t]], gbuf.at[b], gsems[b]
        )

    def _wait_gather(b):
        pltpu.make_async_copy(
            table_hbm.at[pl.ds(0, CHUNK)], gbuf.at[b], gsems[b]
        ).wait()

    def _put(l, b):
        pltpu.async_copy(
            tbuf.at[b],
            out_hbm.at[l, :, wid],
            osems[b],
        )

    def _wait_puts(b):
        pltpu.make_async_copy(
            tbuf.at[b],
            out_hbm.at[0, :, wid],
            osems[b],
        ).wait()

    for b in range(NBUF):
        pltpu.async_copy(out_hbm.at[0, :, wid], dump, osems[b])

    for t in range(NBUF):
        _start_gather(jnp.int32(t), t)

    def trip(i, carry):
        for b in range(NBUF):
            l_raw = i * NBUF + b
            l = jnp.minimum(l_raw, last)
            _wait_gather(b)
            _wait_puts(b)

            mfs = []
            for g in range(CHUNK // 16):
                v = idx_v[l, pl.ds(g * 16, 16)]
                mfs.append(jnp.where(v == 0, 0.0, 1.0).astype(jnp.float32))

            @plsc.parallel_loop(0, DIM // 8, unroll=4)
            def fbody(fq, b=b, mfs=mfs):
                for df in range(8):
                    f = fq * 8 + df
                    col = jnp.full((16,), f, jnp.int32)
                    vals = [
                        plsc.load_gather(gbuf.at[b], [rows_g[g], col])
                        for g in range(CHUNK // 16)
                    ]
                    for g in range(CHUNK // 16):
                        tbuf[b, fq, df, pl.ds(g * 16, 16)] = vals[g] * mfs[g]

            _start_gather(jnp.minimum(l + NBUF, last), b)
            _put(l, b)
        return carry

    lax.fori_loop(0, NTASK_PAD // NBUF, trip, 0, unroll=False)

    for b in range(NBUF):
        _wait_gather(b)
        _wait_puts(b)


def kernel(x, table):
    xt = x.T.astype(jnp.int32)
    tablep = _transpose_table(table.T)
    out5 = _emb_fused(xt, tablep)
    return out5.transpose(2, 4, 0, 1, 3).reshape(B, L, DIM)


# speedup vs baseline: 1.7735x; 1.2453x over previous
import functools

import jax
import jax.numpy as jnp
from jax import lax
from jax.experimental import pallas as pl
from jax.experimental.pallas import tpu as pltpu
from jax.experimental.pallas import tpu_sc as plsc

DIM = 64
B_TOTAL = 4096 * 50
NC = 2
NS = 16
NW = NC * NS
PER_W = B_TOTAL // NW
CHUNK = 128
NCH = PER_W // CHUNK
NBUF = 6
AHEAD = 3

_mesh = plsc.VectorSubcoreMesh(core_axis_name="c", subcore_axis_name="s")


@functools.partial(
    pl.kernel,
    mesh=_mesh,
    out_type=jax.ShapeDtypeStruct((B_TOTAL, DIM), jnp.float32),
    scratch_types=[
        pltpu.VMEM((NCH, CHUNK), jnp.int32),
        pltpu.VMEM((NBUF, CHUNK, DIM), jnp.float32),
        pltpu.VMEM((16, DIM), jnp.float32),
        pltpu.VMEM((PER_W + 16,), jnp.int32),
    ]
    + [pltpu.SemaphoreType.DMA] * (2 * NBUF + 2),
    compiler_params=pltpu.CompilerParams(
        use_tc_tiling_on_sc=False, needs_layout_passes=False
    ),
)
def _emb_gather(x_hbm, table_hbm, out_hbm, idx_v, bufs, zrow, plist, *sems):
    idx_sem = sems[0]
    fix_sem = sems[1]
    gsems = sems[2 : 2 + NBUF]
    psems = sems[2 + NBUF :]
    wid = lax.axis_index("s") * NC + lax.axis_index("c")
    base = wid * PER_W

    pltpu.async_copy(x_hbm.at[wid], idx_v, idx_sem).wait()
    zeros16 = jnp.zeros((16,), jnp.float32)
    for r in range(16):
        for c in range(DIM // 16):
            zrow[r, pl.ds(c * 16, 16)] = zeros16

    hg = [None] * NCH
    hp = [None] * NCH
    for t in range(NCH + AHEAD):
        g = t
        if g < NCH:
            b = g % NBUF
            if g - NBUF >= 0:
                hp[g - NBUF].wait()
            hg[g] = pltpu.async_copy(
                table_hbm.at[idx_v.at[g]], bufs.at[b], gsems[b]
            )
        d = t - AHEAD
        if 0 <= d < NCH:
            b = d % NBUF
            hg[d].wait()
            hp[d] = pltpu.async_copy(
                bufs.at[b], out_hbm.at[pl.ds(base + d * CHUNK, CHUNK)], psems[b]
            )
    for d in range(NCH - NBUF, NCH):
        hp[d].wait()

    lanes = lax.iota(jnp.int32, 16)
    big = jnp.int32(2**30)

    def _compact(i, carry):
        off, first = carry
        d = i // (CHUNK // 16)
        g = i % (CHUNK // 16)
        v = idx_v[d, pl.ds(g * 16, 16)]
        m = v == 0
        pos = base + i * 16 + lanes
        first = jnp.minimum(first, jnp.min(jnp.where(m, pos, big)))
        plsc.store_compressed(plist.at[pl.ds(off, 16)], pos, mask=m)
        cnt = plsc.all_reduce_population_count(m)[0]
        return off + cnt, first

    npad, first = lax.fori_loop(
        0, PER_W // 16, _compact, (jnp.int32(0), big)
    )
    plist[pl.ds(npad, 16)] = jnp.full((16,), first, jnp.int32)

    def _scatter_zeros(j, carry):
        tv = plist[pl.ds(j * 16, 16)]
        pltpu.async_copy(zrow, out_hbm.at[tv], fix_sem).wait()
        return carry

    lax.fori_loop(0, (npad + 15) // 16, _scatter_zeros, 0, unroll=False)


def kernel(x, table):
    xf = x.reshape(NW, NCH, CHUNK).astype(jnp.int32)
    out = _emb_gather(xf, table)
    return out.reshape(x.shape[0], x.shape[1], DIM)
